# Initial kernel scaffold; baseline (speedup 1.0000x reference)
import jax
import jax.numpy as jnp
from jax.experimental import pallas as pl


def kernel(x, edge_index, W1, b1, W2, b2):
    raise NotImplementedError("write your pallas kernel here")


# speedup vs baseline: 45.0053x; 45.0053x over previous
import functools

import jax
import jax.numpy as jnp
from jax import lax
from jax.experimental import pallas as pl
from jax.experimental.pallas import tpu as pltpu
from jax.experimental.pallas import tpu_sc as plsc

NC = 2
NS = 16
NW = NC * NS
CHUNK = 128
K = 8


def _round_up(a, b):
    return (a + b - 1) // b * b


def _sc_mesh():
    return plsc.VectorSubcoreMesh(core_axis_name="c", subcore_axis_name="s",
                                  num_cores=NC, num_subcores=NS)


_SC_PARAMS = pltpu.CompilerParams(use_tc_tiling_on_sc=False)


def _deg_body(nrows_w, tpt, t, dstr, ones_h, z1, degp, deg_sh, ones_v, didx,
              zb):
    c = lax.axis_index("c")
    s = lax.axis_index("s")
    pltpu.sync_copy(z1, zb)
    pltpu.sync_copy(zb, deg_sh.at[pl.ds(s * tpt, tpt)])
    pltpu.sync_copy(ones_h, ones_v)
    plsc.subcore_barrier()
    base = (s * NC + c) * nrows_w

    def grp(g, carry):
        pltpu.sync_copy(dstr.at[pl.ds(base + g * K, K)], didx)
        for j in range(K):
            pltpu.sync_copy(ones_v, deg_sh.at[didx.at[j]], add=True)
        return carry

    lax.fori_loop(0, nrows_w // K, grp, 0)
    plsc.subcore_barrier()
    pltpu.sync_copy(deg_sh.at[pl.ds(s * tpt, tpt)], zb)
    pltpu.sync_copy(zb, degp.at[pl.ds(c * t + s * tpt, tpt)])


def _half_pass(nrows_w, tpt, c, s, u_h, srcr, dstr, z2, outp,
               tab_sh, sidx, didx, rows, gsem, zb):
    hpt = tpt // 2
    pltpu.sync_copy(z2, zb)
    for q in range(2):
        pltpu.sync_copy(zb, tab_sh.at[pl.ds(s * tpt + q * hpt, hpt)])
    plsc.subcore_barrier()
    base = (s * NC + c) * nrows_w

    def grp(g, carry):
        pltpu.sync_copy(srcr.at[pl.ds(base + g * K, K)], sidx)
        pltpu.sync_copy(dstr.at[pl.ds(base + g * K, K)], didx)
        descs = [pltpu.async_copy(u_h.at[sidx.at[j]], rows.at[j], gsem)
                 for j in range(K)]
        for d in descs:
            d.wait()
        for j in range(K):
            pltpu.sync_copy(rows.at[j], tab_sh.at[didx.at[j]], add=True)
        return carry

    lax.fori_loop(0, nrows_w // K, grp, 0)
    plsc.subcore_barrier()
    for q in range(2):
        pltpu.sync_copy(tab_sh.at[pl.ds(s * tpt + q * hpt, hpt)], zb)
        pltpu.sync_copy(zb, outp.at[c, pl.ds(s * tpt + q * hpt, hpt)])
    plsc.subcore_barrier()


def _layer1_body(nrows_w, tpt, ua, ub, srcr, dstr, z2, outa, outb,
                 tab_sh, sidx, didx, rows, gsem, zb):
    c = lax.axis_index("c")
    s = lax.axis_index("s")
    _half_pass(nrows_w, tpt, c, s, ua, srcr, dstr, z2, outa,
               tab_sh, sidx, didx, rows, gsem, zb)
    _half_pass(nrows_w, tpt, c, s, ub, srcr, dstr, z2, outb,
               tab_sh, sidx, didx, rows, gsem, zb)


def _layer2_body(nrows_w, tpt, u, srcr, dstr, z2, outp,
                 tab_sh, sidx, didx, rows, gsem, zb):
    c = lax.axis_index("c")
    s = lax.axis_index("s")
    _half_pass(nrows_w, tpt, c, s, u, srcr, dstr, z2, outp,
               tab_sh, sidx, didx, rows, gsem, zb)


def _dense1_body(dp0, dp1, x_r, w1_r, ua_r, ub_r):
    dinv = lax.rsqrt(dp0[...] + dp1[...] + 1.0)
    h = jnp.dot(x_r[...], w1_r[...], preferred_element_type=jnp.float32)
    u = h * dinv
    fh = u.shape[1] // 2
    ua_r[...] = u[:, :fh]
    ub_r[...] = u[:, fh:]


def _dense2_body(a0a, a1a, a0b, a1b, ua, ub, dp0, dp1, b1a_r, b1b_r,
                 w2a_r, w2b_r, u2_r):
    dinv = lax.rsqrt(dp0[...] + dp1[...] + 1.0)
    ha = jnp.maximum(dinv * (a0a[...] + a1a[...] + ua[...]) + b1a_r[...], 0.0)
    hb = jnp.maximum(dinv * (a0b[...] + a1b[...] + ub[...]) + b1b_r[...], 0.0)
    h2 = (jnp.dot(ha, w2a_r[...], preferred_element_type=jnp.float32)
          + jnp.dot(hb, w2b_r[...], preferred_element_type=jnp.float32))
    u2_r[...] = h2 * dinv


def _dense3_body(a0, a1, u2, dp0, dp1, b2_r, out_r):
    dinv = lax.rsqrt(dp0[...] + dp1[...] + 1.0)
    out_r[...] = dinv * (a0[...] + a1[...] + u2[...]) + b2_r[...]


def kernel(x, edge_index, W1, b1, W2, b2):
    n, f0 = x.shape
    f1 = W1.shape[1]
    f2 = W2.shape[1]
    fh = f1 // 2
    e = edge_index.shape[1]

    nrows_w = _round_up(-(-e // (NW * CHUNK)), K)
    ep = NW * nrows_w * CHUNK
    t = _round_up(n + 1, NS * 8)
    tpt = t // NS

    src = edge_index[0]
    dst = edge_index[1]
    pad = ep - e
    srcp = jnp.concatenate([src, jnp.zeros((pad,), jnp.int32)]
                           ).reshape(ep // CHUNK, CHUNK)
    dstp = jnp.concatenate([dst, jnp.full((pad,), n, jnp.int32)]
                           ).reshape(ep // CHUNK, CHUNK)

    deg_call = pl.kernel(
        functools.partial(_deg_body, nrows_w, tpt, t),
        out_type=jax.ShapeDtypeStruct((NC * t,), jnp.float32),
        mesh=_sc_mesh(),
        scratch_types=[
            pltpu.VMEM_SHARED((t,), jnp.float32),
            pltpu.VMEM((CHUNK,), jnp.float32),
            pltpu.VMEM((K, CHUNK), jnp.int32),
            pltpu.VMEM((tpt,), jnp.float32),
        ],
        compiler_params=_SC_PARAMS,
    )
    degp = deg_call(dstp, jnp.ones((CHUNK,), jnp.float32),
                    jnp.zeros((tpt,), jnp.float32)).reshape(NC, t)
    dp0 = degp[0, :n, None]
    dp1 = degp[1, :n, None]

    bn = 5000
    grid = (n // bn,)
    col = pl.BlockSpec((bn, 1), lambda i: (i, 0))

    def half_spec():
        return pl.BlockSpec((bn, fh), lambda i: (i, 0))

    u1a, u1b = pl.pallas_call(
        _dense1_body,
        grid=grid,
        in_specs=[col, col,
                  pl.BlockSpec((bn, f0), lambda i: (i, 0)),
                  pl.BlockSpec((f0, f1), lambda i: (0, 0))],
        out_specs=[half_spec(), half_spec()],
        out_shape=[jax.ShapeDtypeStruct((n, fh), jnp.float32),
                   jax.ShapeDtypeStruct((n, fh), jnp.float32)],
    )(dp0, dp1, x, W1)

    layer_scratch = [
        pltpu.VMEM_SHARED((t, fh), jnp.float32),
        pltpu.VMEM((K, CHUNK), jnp.int32),
        pltpu.VMEM((K, CHUNK), jnp.int32),
        pltpu.VMEM((K, CHUNK, fh), jnp.float32),
        pltpu.SemaphoreType.DMA,
        pltpu.VMEM((tpt // 2, fh), jnp.float32),
    ]
    z2 = jnp.zeros((tpt // 2, fh), jnp.float32)

    l1_call = pl.kernel(
        functools.partial(_layer1_body, nrows_w, tpt),
        out_type=[jax.ShapeDtypeStruct((NC, t, fh), jnp.float32),
                  jax.ShapeDtypeStruct((NC, t, fh), jnp.float32)],
        mesh=_sc_mesh(),
        scratch_types=layer_scratch,
        compiler_params=_SC_PARAMS,
    )
    acc1a, acc1b = l1_call(u1a, u1b, srcp, dstp, z2)

    u2 = pl.pallas_call(
        _dense2_body,
        grid=grid,
        in_specs=[half_spec(), half_spec(), half_spec(), half_spec(),
                  half_spec(), half_spec(), col, col,
                  pl.BlockSpec((1, fh), lambda i: (0, 0)),
                  pl.BlockSpec((1, fh), lambda i: (0, 0)),
                  pl.BlockSpec((fh, f2), lambda i: (0, 0)),
                  pl.BlockSpec((fh, f2), lambda i: (0, 0))],
        out_specs=pl.BlockSpec((bn, f2), lambda i: (i, 0)),
        out_shape=jax.ShapeDtypeStruct((n, f2), jnp.float32),
    )(acc1a[0, :n], acc1a[1, :n], acc1b[0, :n], acc1b[1, :n], u1a, u1b,
      dp0, dp1, b1[:fh].reshape(1, fh), b1[fh:].reshape(1, fh),
      W2[:fh], W2[fh:])

    l2_call = pl.kernel(
        functools.partial(_layer2_body, nrows_w, tpt),
        out_type=jax.ShapeDtypeStruct((NC, t, f2), jnp.float32),
        mesh=_sc_mesh(),
        scratch_types=layer_scratch,
        compiler_params=_SC_PARAMS,
    )
    acc2 = l2_call(u2, srcp, dstp, z2)

    out = pl.pallas_call(
        _dense3_body,
        grid=grid,
        in_specs=[pl.BlockSpec((bn, f2), lambda i: (i, 0)),
                  pl.BlockSpec((bn, f2), lambda i: (i, 0)),
                  pl.BlockSpec((bn, f2), lambda i: (i, 0)),
                  col, col,
                  pl.BlockSpec((1, f2), lambda i: (0, 0))],
        out_specs=pl.BlockSpec((bn, f2), lambda i: (i, 0)),
        out_shape=jax.ShapeDtypeStruct((n, f2), jnp.float32),
    )(acc2[0, :n], acc2[1, :n], u2, dp0, dp1, b2.reshape(1, f2))
    return out
